# baseline (speedup 1.0000x reference)
import jax
import jax.numpy as jnp
from jax import lax
from jax.experimental import pallas as pl
from jax.experimental.pallas import tpu as pltpu
from jax.experimental.pallas import tpu_sc as plsc

F32 = jnp.float32
I32 = jnp.int32

N_PEDS = 10000
NNZ = 80000
FEAT = 96
ATT_DIM = 32
OUT_C = 128
NPAIRS = 8

NC = 2
NS = 16
PAIRS_PER_CORE = NPAIRS // NC
NPAD = 10240
ROWS_PER_TILE = NPAD // NS
CHUNK = 112
NCHUNK = 46
NNZ_TILE = CHUNK * NCHUNK
CCHUNK = NNZ_TILE // 2
DUMMY = NPAD - 1
RCHUNK = 32
NRCHUNK = ROWS_PER_TILE // RCHUNK
NLANE = 16


def _tc_matmul_body(x_ref, att_ref, wtop_ref, wbot_ref, out_ref):
    xb = x_ref[0, 0]
    ab = att_ref[0, 0]
    top = lax.dot_general(xb, wtop_ref[...], (((0,), (0,)), ((), ())),
                          preferred_element_type=F32)
    bot = lax.dot_general(ab, wbot_ref[...], (((1,), (0,)), ((), ())),
                          preferred_element_type=F32)
    out_ref[0, pl.ds(0, top.shape[0]), :] = top + bot


def _tc_matmul(xt, att, lin_w):
    b, t, f, n = xt.shape
    wtop = lin_w[:FEAT]
    wbot = lin_w[FEAT:]
    return pl.pallas_call(
        _tc_matmul_body,
        grid=(NPAIRS,),
        in_specs=[
            pl.BlockSpec((1, 1, FEAT, n),
                         lambda p: (lax.div(p, 4), lax.rem(p, 4), 0, 0)),
            pl.BlockSpec((1, 1, n, ATT_DIM),
                         lambda p: (lax.div(p, 4), lax.rem(p, 4), 0, 0)),
            pl.BlockSpec((FEAT, OUT_C), lambda p: (0, 0)),
            pl.BlockSpec((ATT_DIM, OUT_C), lambda p: (0, 0)),
        ],
        out_specs=pl.BlockSpec((1, NPAD, OUT_C), lambda p: (p, 0, 0)),
        out_shape=jax.ShapeDtypeStruct((NPAIRS, NPAD, OUT_C), F32),
    )(xt, att, wtop, wbot)


def _tc_scale_body(raw_ref, dcnt_ref, bias_ref, out_ref):
    n = out_ref.shape[1]
    d = dcnt_ref[0, 0, :]
    inv = 1.0 / jnp.maximum(d, 1.0)
    out_ref[0] = (raw_ref[0, :n, :] * inv[:n, None]) + bias_ref[0][None, :]


def _tc_scale(raw, dcnt, bias, n):
    return pl.pallas_call(
        _tc_scale_body,
        grid=(NPAIRS,),
        in_specs=[
            pl.BlockSpec((1, NPAD, OUT_C), lambda p: (p, 0, 0)),
            pl.BlockSpec((1, 1, NPAD), lambda p: (p, 0, 0)),
            pl.BlockSpec((1, OUT_C), lambda p: (0, 0)),
        ],
        out_specs=pl.BlockSpec((1, n, OUT_C), lambda p: (p, 0, 0)),
        out_shape=jax.ShapeDtypeStruct((NPAIRS, n, OUT_C), F32),
    )(raw, dcnt.reshape(NPAIRS, 1, NPAD), bias.reshape(1, OUT_C))


def _sc_body(xl3, nidx, eidx, zeros2d, zeros1d,
             out_h, dcnt_h, edge_h,
             table, bcnt, dcnt,
             nidx_v, eil_v, rowbuf, ones_c, rowchunk, cnt_v,
             sem0):
    c = lax.axis_index("c")
    s = lax.axis_index("s")
    r0 = s * ROWS_PER_TILE

    ones16 = jnp.ones((NLANE,), F32)

    @pl.loop(0, ones_c.shape[0] // NLANE)
    def _(i):
        ones_c[pl.ds(i * NLANE, NLANE)] = ones16

    pltpu.sync_copy(zeros2d.at[pl.ds(r0, ROWS_PER_TILE)],
                    table.at[pl.ds(r0, ROWS_PER_TILE)])
    pltpu.sync_copy(zeros1d.at[pl.ds(r0, ROWS_PER_TILE)],
                    bcnt.at[pl.ds(r0, ROWS_PER_TILE)])
    pltpu.sync_copy(zeros1d.at[pl.ds(r0, ROWS_PER_TILE)],
                    dcnt.at[pl.ds(r0, ROWS_PER_TILE)])
    plsc.subcore_barrier()

    @pl.loop(0, PAIRS_PER_CORE)
    def _(q):
        p = c * PAIRS_PER_CORE + q
        xlp = xl3.at[p]
        ehp = edge_h.at[p]

        pltpu.sync_copy(nidx.at[p, s], nidx_v)
        pltpu.sync_copy(eidx.at[p, s], eil_v)

        for half in range(2):
            pltpu.sync_copy(ones_c,
                            bcnt.at[eil_v.at[pl.ds(half * CCHUNK, CCHUNK)]],
                            add=True)
            pltpu.sync_copy(ones_c,
                            dcnt.at[nidx_v.at[pl.ds(half * CCHUNK, CCHUNK)]],
                            add=True)

        @pl.loop(0, NCHUNK)
        def _(j):
            pltpu.async_copy(
                xlp.at[nidx_v.at[pl.ds(j * CHUNK, CHUNK)]], rowbuf, sem0
            ).wait()
            pltpu.sync_copy(rowbuf,
                            table.at[eil_v.at[pl.ds(j * CHUNK, CHUNK)]],
                            add=True)

        plsc.subcore_barrier()

        pltpu.sync_copy(bcnt.at[pl.ds(r0, ROWS_PER_TILE)], cnt_v)

        @pl.loop(0, ROWS_PER_TILE // NLANE)
        def _(k):
            cv = cnt_v[pl.ds(k * NLANE, NLANE)]
            cnt_v[pl.ds(k * NLANE, NLANE)] = 1.0 / jnp.maximum(cv, 1.0)

        @pl.loop(0, NRCHUNK)
        def _(i):
            pltpu.sync_copy(table.at[pl.ds(r0 + i * RCHUNK, RCHUNK)], rowchunk)

            @pl.loop(0, RCHUNK // NLANE)
            def _(g):
                invv = cnt_v[pl.ds(i * RCHUNK + g * NLANE, NLANE)]
                for r in range(NLANE):
                    inv = jnp.full((NLANE,), invv[r], F32)
                    row = g * NLANE + r
                    for cc in range(OUT_C // NLANE):
                        rowchunk[row, pl.ds(cc * NLANE, NLANE)] = (
                            rowchunk[row, pl.ds(cc * NLANE, NLANE)] * inv)

            pltpu.sync_copy(rowchunk, ehp.at[pl.ds(r0 + i * RCHUNK, RCHUNK)])

        pltpu.sync_copy(zeros2d.at[pl.ds(r0, ROWS_PER_TILE)],
                        table.at[pl.ds(r0, ROWS_PER_TILE)])
        pltpu.sync_copy(zeros1d.at[pl.ds(r0, ROWS_PER_TILE)],
                        bcnt.at[pl.ds(r0, ROWS_PER_TILE)])
        plsc.subcore_barrier()

        @pl.loop(0, NCHUNK)
        def _(j):
            pltpu.async_copy(
                ehp.at[eil_v.at[pl.ds(j * CHUNK, CHUNK)]], rowbuf, sem0
            ).wait()
            pltpu.sync_copy(rowbuf,
                            table.at[nidx_v.at[pl.ds(j * CHUNK, CHUNK)]],
                            add=True)

        plsc.subcore_barrier()

        pltpu.sync_copy(table.at[pl.ds(r0, ROWS_PER_TILE)],
                        out_h.at[p].at[pl.ds(r0, ROWS_PER_TILE)])
        pltpu.sync_copy(dcnt.at[pl.ds(r0, ROWS_PER_TILE)],
                        dcnt_h.at[p].at[pl.ds(r0, ROWS_PER_TILE)])
        pltpu.sync_copy(zeros2d.at[pl.ds(r0, ROWS_PER_TILE)],
                        table.at[pl.ds(r0, ROWS_PER_TILE)])
        pltpu.sync_copy(zeros1d.at[pl.ds(r0, ROWS_PER_TILE)],
                        dcnt.at[pl.ds(r0, ROWS_PER_TILE)])
        plsc.subcore_barrier()


def _sc_hyperconv(xl3, nidx, eidx, zeros2d, zeros1d):
    mesh = plsc.VectorSubcoreMesh(core_axis_name="c", subcore_axis_name="s",
                                  num_cores=NC, num_subcores=NS)
    f = pl.kernel(
        _sc_body,
        out_type=(
            jax.ShapeDtypeStruct((NPAIRS, NPAD, OUT_C), F32),
            jax.ShapeDtypeStruct((NPAIRS, NPAD), F32),
            jax.ShapeDtypeStruct((NPAIRS, NPAD, OUT_C), F32),
        ),
        mesh=mesh,
        scratch_types=[
            pltpu.VMEM_SHARED((NPAD, OUT_C), F32),
            pltpu.VMEM_SHARED((NPAD,), F32),
            pltpu.VMEM_SHARED((NPAD,), F32),
            pltpu.VMEM((NNZ_TILE,), I32),
            pltpu.VMEM((NNZ_TILE,), I32),
            pltpu.VMEM((CHUNK, OUT_C), F32),
            pltpu.VMEM((CCHUNK,), F32),
            pltpu.VMEM((RCHUNK, OUT_C), F32),
            pltpu.VMEM((ROWS_PER_TILE,), F32),
            pltpu.SemaphoreType.DMA,
        ],
    )
    return f(xl3, nidx, eidx, zeros2d, zeros1d)


@jax.jit
def kernel(x, H, sequential_scene_attention, W, lin_w, bias):
    b, f, t, n = x.shape
    xt = jnp.transpose(x, (0, 2, 1, 3))
    xl = _tc_matmul(xt, sequential_scene_attention, lin_w)

    node = H[:, :, 0, :].reshape(NPAIRS, NS, NNZ // NS)
    edge = H[:, :, 1, :].reshape(NPAIRS, NS, NNZ // NS)
    pad = ((0, 0), (0, 0), (0, NNZ_TILE - NNZ // NS))
    nidx = jnp.pad(node, pad, constant_values=DUMMY)
    eidx = jnp.pad(edge, pad, constant_values=DUMMY)
    zeros2d = jnp.zeros((NPAD, OUT_C), F32)
    zeros1d = jnp.zeros((NPAD,), F32)

    raw, dcnt, _ = _sc_hyperconv(xl, nidx, eidx, zeros2d, zeros1d)
    out = _tc_scale(raw, dcnt, bias, n)
    return out.reshape(b, OUT_C, t, n)


# speedup vs baseline: 1.0895x; 1.0895x over previous
import jax
import jax.numpy as jnp
from jax import lax
from jax.experimental import pallas as pl
from jax.experimental.pallas import tpu as pltpu
from jax.experimental.pallas import tpu_sc as plsc

F32 = jnp.float32
I32 = jnp.int32

N_PEDS = 10000
NNZ = 80000
FEAT = 96
ATT_DIM = 32
OUT_C = 128
NPAIRS = 8

NC = 2
NS = 16
PAIRS_PER_CORE = NPAIRS // NC
NPAD = 10240
ROWS_PER_TILE = NPAD // NS
CHUNK = 128
NCHUNK = 40
NNZ_TILE = CHUNK * NCHUNK
CCHUNK = NNZ_TILE // 2
DUMMY = NPAD - 1
RCHUNK = 32
NRCHUNK = ROWS_PER_TILE // RCHUNK
NLANE = 16


def _tc_matmul_body(x_ref, att_ref, wtop_ref, wbot_ref, out_ref):
    xb = x_ref[0, 0]
    ab = att_ref[0, 0]
    top = lax.dot_general(xb, wtop_ref[...], (((0,), (0,)), ((), ())),
                          preferred_element_type=F32)
    bot = lax.dot_general(ab, wbot_ref[...], (((1,), (0,)), ((), ())),
                          preferred_element_type=F32)
    out_ref[0, pl.ds(0, top.shape[0]), :] = top + bot


def _tc_matmul(xt, att, lin_w):
    b, t, f, n = xt.shape
    wtop = lin_w[:FEAT]
    wbot = lin_w[FEAT:]
    return pl.pallas_call(
        _tc_matmul_body,
        grid=(NPAIRS,),
        in_specs=[
            pl.BlockSpec((1, 1, FEAT, n),
                         lambda p: (lax.div(p, 4), lax.rem(p, 4), 0, 0)),
            pl.BlockSpec((1, 1, n, ATT_DIM),
                         lambda p: (lax.div(p, 4), lax.rem(p, 4), 0, 0)),
            pl.BlockSpec((FEAT, OUT_C), lambda p: (0, 0)),
            pl.BlockSpec((ATT_DIM, OUT_C), lambda p: (0, 0)),
        ],
        out_specs=pl.BlockSpec((1, NPAD, OUT_C), lambda p: (p, 0, 0)),
        out_shape=jax.ShapeDtypeStruct((NPAIRS, NPAD, OUT_C), F32),
    )(xt, att, wtop, wbot)


def _tc_scale_body(raw_ref, dcnt_ref, bias_ref, out_ref):
    n = out_ref.shape[1]
    d = dcnt_ref[0, 0, :]
    inv = 1.0 / jnp.maximum(d, 1.0)
    out_ref[0] = (raw_ref[0, :n, :] * inv[:n, None]) + bias_ref[0][None, :]


def _tc_scale(raw, dcnt, bias, n):
    return pl.pallas_call(
        _tc_scale_body,
        grid=(NPAIRS,),
        in_specs=[
            pl.BlockSpec((1, NPAD, OUT_C), lambda p: (p, 0, 0)),
            pl.BlockSpec((1, 1, NPAD), lambda p: (p, 0, 0)),
            pl.BlockSpec((1, OUT_C), lambda p: (0, 0)),
        ],
        out_specs=pl.BlockSpec((1, n, OUT_C), lambda p: (p, 0, 0)),
        out_shape=jax.ShapeDtypeStruct((NPAIRS, n, OUT_C), F32),
    )(raw, dcnt.reshape(NPAIRS, 1, NPAD), bias.reshape(1, OUT_C))


def _sc_body(xlf, nidx, eidx, zeros2d, zeros1d,
             out_h, dcnt_h, edge_h,
             table, bcnt, dcnt,
             nidx_v, eil_v, rowbuf, ones_c, rowchunk, cnt_v,
             sem0):
    c = lax.axis_index("c")
    s = lax.axis_index("s")
    r0 = s * ROWS_PER_TILE

    ones16 = jnp.ones((NLANE,), F32)

    @pl.loop(0, ones_c.shape[0] // NLANE)
    def _(i):
        ones_c[pl.ds(i * NLANE, NLANE)] = ones16

    pltpu.sync_copy(zeros2d.at[pl.ds(r0, ROWS_PER_TILE)],
                    table.at[pl.ds(r0, ROWS_PER_TILE)])
    pltpu.sync_copy(zeros1d.at[pl.ds(r0, ROWS_PER_TILE)],
                    bcnt.at[pl.ds(r0, ROWS_PER_TILE)])
    pltpu.sync_copy(zeros1d.at[pl.ds(r0, ROWS_PER_TILE)],
                    dcnt.at[pl.ds(r0, ROWS_PER_TILE)])
    plsc.subcore_barrier()

    @pl.loop(0, PAIRS_PER_CORE)
    def _(q):
        p = c * PAIRS_PER_CORE + q
        base = p * NPAD

        pltpu.sync_copy(nidx.at[p, s], nidx_v)
        pltpu.sync_copy(eidx.at[p, s], eil_v)

        for half in range(2):
            pltpu.sync_copy(ones_c,
                            bcnt.at[eil_v.at[pl.ds(half * CCHUNK, CCHUNK)]],
                            add=True)

        @pl.loop(0, NCHUNK)
        def _(j):
            pltpu.async_copy(
                xlf.at[nidx_v.at[pl.ds(j * CHUNK, CHUNK)]], rowbuf, sem0
            ).wait()
            pltpu.sync_copy(rowbuf,
                            table.at[eil_v.at[pl.ds(j * CHUNK, CHUNK)]],
                            add=True)

        off16 = jnp.full((NLANE,), base, I32)

        @pl.loop(0, NNZ_TILE // NLANE)
        def _(i):
            nidx_v[pl.ds(i * NLANE, NLANE)] = (
                nidx_v[pl.ds(i * NLANE, NLANE)] - off16)
            eil_v[pl.ds(i * NLANE, NLANE)] = (
                eil_v[pl.ds(i * NLANE, NLANE)] + off16)

        plsc.subcore_barrier()

        pltpu.sync_copy(bcnt.at[pl.ds(r0, ROWS_PER_TILE)], cnt_v)

        @pl.loop(0, ROWS_PER_TILE // NLANE)
        def _(k):
            cv = cnt_v[pl.ds(k * NLANE, NLANE)]
            cnt_v[pl.ds(k * NLANE, NLANE)] = 1.0 / jnp.maximum(cv, 1.0)

        @pl.loop(0, NRCHUNK)
        def _(i):
            pltpu.sync_copy(table.at[pl.ds(r0 + i * RCHUNK, RCHUNK)], rowchunk)

            @pl.loop(0, RCHUNK // NLANE)
            def _(g):
                invv = cnt_v[pl.ds(i * RCHUNK + g * NLANE, NLANE)]
                for r in range(NLANE):
                    inv = jnp.full((NLANE,), invv[r], F32)
                    row = g * NLANE + r
                    for cc in range(OUT_C // NLANE):
                        rowchunk[row, pl.ds(cc * NLANE, NLANE)] = (
                            rowchunk[row, pl.ds(cc * NLANE, NLANE)] * inv)

            pltpu.sync_copy(
                rowchunk, edge_h.at[pl.ds(base + r0 + i * RCHUNK, RCHUNK)])

        pltpu.sync_copy(zeros2d.at[pl.ds(r0, ROWS_PER_TILE)],
                        table.at[pl.ds(r0, ROWS_PER_TILE)])
        pltpu.sync_copy(zeros1d.at[pl.ds(r0, ROWS_PER_TILE)],
                        bcnt.at[pl.ds(r0, ROWS_PER_TILE)])
        plsc.subcore_barrier()

        for half in range(2):
            pltpu.sync_copy(ones_c,
                            dcnt.at[nidx_v.at[pl.ds(half * CCHUNK, CCHUNK)]],
                            add=True)

        @pl.loop(0, NCHUNK)
        def _(j):
            pltpu.async_copy(
                edge_h.at[eil_v.at[pl.ds(j * CHUNK, CHUNK)]], rowbuf, sem0
            ).wait()
            pltpu.sync_copy(rowbuf,
                            table.at[nidx_v.at[pl.ds(j * CHUNK, CHUNK)]],
                            add=True)

        plsc.subcore_barrier()

        pltpu.sync_copy(table.at[pl.ds(r0, ROWS_PER_TILE)],
                        out_h.at[pl.ds(base + r0, ROWS_PER_TILE)])
        pltpu.sync_copy(dcnt.at[pl.ds(r0, ROWS_PER_TILE)],
                        dcnt_h.at[pl.ds(base + r0, ROWS_PER_TILE)])
        pltpu.sync_copy(zeros2d.at[pl.ds(r0, ROWS_PER_TILE)],
                        table.at[pl.ds(r0, ROWS_PER_TILE)])
        pltpu.sync_copy(zeros1d.at[pl.ds(r0, ROWS_PER_TILE)],
                        dcnt.at[pl.ds(r0, ROWS_PER_TILE)])
        plsc.subcore_barrier()


def _sc_hyperconv(xlf, nidx, eidx, zeros2d, zeros1d):
    mesh = plsc.VectorSubcoreMesh(core_axis_name="c", subcore_axis_name="s",
                                  num_cores=NC, num_subcores=NS)
    f = pl.kernel(
        _sc_body,
        out_type=(
            jax.ShapeDtypeStruct((NPAIRS * NPAD, OUT_C), F32),
            jax.ShapeDtypeStruct((NPAIRS * NPAD,), F32),
            jax.ShapeDtypeStruct((NPAIRS * NPAD, OUT_C), F32),
        ),
        mesh=mesh,
        scratch_types=[
            pltpu.VMEM_SHARED((NPAD, OUT_C), F32),
            pltpu.VMEM_SHARED((NPAD,), F32),
            pltpu.VMEM_SHARED((NPAD,), F32),
            pltpu.VMEM((NNZ_TILE,), I32),
            pltpu.VMEM((NNZ_TILE,), I32),
            pltpu.VMEM((CHUNK, OUT_C), F32),
            pltpu.VMEM((CCHUNK,), F32),
            pltpu.VMEM((RCHUNK, OUT_C), F32),
            pltpu.VMEM((ROWS_PER_TILE,), F32),
            pltpu.SemaphoreType.DMA,
        ],
    )
    return f(xlf, nidx, eidx, zeros2d, zeros1d)


@jax.jit
def kernel(x, H, sequential_scene_attention, W, lin_w, bias):
    b, f, t, n = x.shape
    xt = jnp.transpose(x, (0, 2, 1, 3))
    xl = _tc_matmul(xt, sequential_scene_attention, lin_w)

    node = H[:, :, 0, :].reshape(NPAIRS, NS, NNZ // NS)
    edge = H[:, :, 1, :].reshape(NPAIRS, NS, NNZ // NS)
    pad = ((0, 0), (0, 0), (0, NNZ_TILE - NNZ // NS))
    poff = (jnp.arange(NPAIRS, dtype=I32) * NPAD).reshape(NPAIRS, 1, 1)
    nidx = jnp.pad(node, pad, constant_values=DUMMY) + poff
    eidx = jnp.pad(edge, pad, constant_values=DUMMY)
    zeros2d = jnp.zeros((NPAD, OUT_C), F32)
    zeros1d = jnp.zeros((NPAD,), F32)

    raw, dcnt, _ = _sc_hyperconv(xl.reshape(NPAIRS * NPAD, OUT_C),
                                 nidx, eidx, zeros2d, zeros1d)
    out = _tc_scale(raw.reshape(NPAIRS, NPAD, OUT_C),
                    dcnt.reshape(NPAIRS, NPAD), bias, n)
    return out.reshape(b, OUT_C, t, n)


# speedup vs baseline: 1.7477x; 1.6040x over previous
import jax
import jax.numpy as jnp
from jax import lax
from jax.experimental import pallas as pl
from jax.experimental.pallas import tpu as pltpu
from jax.experimental.pallas import tpu_sc as plsc

F32 = jnp.float32
I32 = jnp.int32

N_PEDS = 10000
NNZ = 80000
FEAT = 96
ATT_DIM = 32
OUT_C = 128
NPAIRS = 8

NC = 2
NS = 16
PAIRS_PER_CORE = NPAIRS // NC
NPAD = 10240
ROWS_PER_TILE = NPAD // NS
NNZ_TILE = NNZ // NS
CHUNK = 128
NCHUNK = NNZ_TILE // CHUNK
TAIL = NNZ_TILE - NCHUNK * CHUNK
CC0 = 2496
CC1 = NNZ_TILE - CC0
RCHUNK = 32
NRCHUNK = ROWS_PER_TILE // RCHUNK
NLANE = 16


def _tc_matmul_body(x_ref, att_ref, wtop_ref, wbot_ref, out_ref):
    xb = x_ref[0, 0]
    ab = att_ref[0, 0]
    top = lax.dot_general(xb, wtop_ref[...], (((0,), (0,)), ((), ())),
                          preferred_element_type=F32)
    bot = lax.dot_general(ab, wbot_ref[...], (((1,), (0,)), ((), ())),
                          preferred_element_type=F32)
    out_ref[0, pl.ds(0, top.shape[0]), :] = top + bot


def _tc_matmul(xt, att, lin_w):
    b, t, f, n = xt.shape
    wtop = lin_w[:FEAT]
    wbot = lin_w[FEAT:]
    return pl.pallas_call(
        _tc_matmul_body,
        grid=(NPAIRS,),
        in_specs=[
            pl.BlockSpec((1, 1, FEAT, n),
                         lambda p: (lax.div(p, 4), lax.rem(p, 4), 0, 0)),
            pl.BlockSpec((1, 1, n, ATT_DIM),
                         lambda p: (lax.div(p, 4), lax.rem(p, 4), 0, 0)),
            pl.BlockSpec((FEAT, OUT_C), lambda p: (0, 0)),
            pl.BlockSpec((ATT_DIM, OUT_C), lambda p: (0, 0)),
        ],
        out_specs=pl.BlockSpec((1, NPAD, OUT_C), lambda p: (p, 0, 0)),
        out_shape=jax.ShapeDtypeStruct((NPAIRS, NPAD, OUT_C), F32),
    )(xt, att, wtop, wbot)


def _tc_scale_body(raw_ref, dcnt_ref, bias_ref, out_ref):
    n = out_ref.shape[1]
    d = dcnt_ref[0, 0, :]
    inv = 1.0 / jnp.maximum(d, 1.0)
    out_ref[0] = (raw_ref[0, :n, :] * inv[:n, None]) + bias_ref[0][None, :]


def _tc_scale(raw, dcnt, bias, n):
    return pl.pallas_call(
        _tc_scale_body,
        grid=(NPAIRS,),
        in_specs=[
            pl.BlockSpec((1, NPAD, OUT_C), lambda p: (p, 0, 0)),
            pl.BlockSpec((1, 1, NPAD), lambda p: (p, 0, 0)),
            pl.BlockSpec((1, OUT_C), lambda p: (0, 0)),
        ],
        out_specs=pl.BlockSpec((1, n, OUT_C), lambda p: (p, 0, 0)),
        out_shape=jax.ShapeDtypeStruct((NPAIRS, n, OUT_C), F32),
    )(raw, dcnt.reshape(NPAIRS, 1, NPAD), bias.reshape(1, OUT_C))


def _sc_body(xlf, nidx, eidx, zeros2d, zeros1d,
             out_h, dcnt_h, edge_h,
             table, bcnt, dcnt,
             nidx_v, eil_v, rowbuf, ones_c, rowchunk, cnt_v,
             sem0):
    c = lax.axis_index("c")
    s = lax.axis_index("s")
    r0 = s * ROWS_PER_TILE

    ones16 = jnp.ones((NLANE,), F32)

    @pl.loop(0, ones_c.shape[0] // NLANE)
    def _(i):
        ones_c[pl.ds(i * NLANE, NLANE)] = ones16

    pltpu.sync_copy(zeros2d.at[pl.ds(r0, ROWS_PER_TILE)],
                    table.at[pl.ds(r0, ROWS_PER_TILE)])
    pltpu.sync_copy(zeros1d.at[pl.ds(r0, ROWS_PER_TILE)],
                    bcnt.at[pl.ds(r0, ROWS_PER_TILE)])
    pltpu.sync_copy(zeros1d.at[pl.ds(r0, ROWS_PER_TILE)],
                    dcnt.at[pl.ds(r0, ROWS_PER_TILE)])
    plsc.subcore_barrier()

    @pl.loop(0, PAIRS_PER_CORE)
    def _(q):
        p = c * PAIRS_PER_CORE + q
        base = p * NPAD

        pltpu.sync_copy(nidx.at[p, s], nidx_v)
        pltpu.sync_copy(eidx.at[p, s], eil_v)

        pltpu.sync_copy(ones_c.at[pl.ds(0, CC0)],
                        bcnt.at[eil_v.at[pl.ds(0, CC0)]], add=True)
        pltpu.sync_copy(ones_c.at[pl.ds(0, CC1)],
                        bcnt.at[eil_v.at[pl.ds(CC0, CC1)]], add=True)

        @pl.loop(0, NCHUNK)
        def _(j):
            pltpu.async_copy(
                xlf.at[nidx_v.at[pl.ds(j * CHUNK, CHUNK)]], rowbuf, sem0
            ).wait()
            pltpu.sync_copy(rowbuf,
                            table.at[eil_v.at[pl.ds(j * CHUNK, CHUNK)]],
                            add=True)

        pltpu.async_copy(
            xlf.at[nidx_v.at[pl.ds(NCHUNK * CHUNK, TAIL)]],
            rowbuf.at[pl.ds(0, TAIL)], sem0).wait()
        pltpu.sync_copy(rowbuf.at[pl.ds(0, TAIL)],
                        table.at[eil_v.at[pl.ds(NCHUNK * CHUNK, TAIL)]],
                        add=True)

        off16 = jnp.full((NLANE,), base, I32)

        @pl.loop(0, NNZ_TILE // NLANE)
        def _(i):
            nidx_v[pl.ds(i * NLANE, NLANE)] = (
                nidx_v[pl.ds(i * NLANE, NLANE)] - off16)
            eil_v[pl.ds(i * NLANE, NLANE)] = (
                eil_v[pl.ds(i * NLANE, NLANE)] + off16)

        tmask = lax.iota(I32, NLANE) >= 2 * NLANE - (NNZ_TILE % NLANE) - NLANE
        tbase = NNZ_TILE - NLANE
        nv = nidx_v[pl.ds(tbase, NLANE)]
        nidx_v[pl.ds(tbase, NLANE)] = jnp.where(tmask, nv - off16, nv)
        ev = eil_v[pl.ds(tbase, NLANE)]
        eil_v[pl.ds(tbase, NLANE)] = jnp.where(tmask, ev + off16, ev)

        plsc.subcore_barrier()

        pltpu.sync_copy(bcnt.at[pl.ds(r0, ROWS_PER_TILE)], cnt_v)

        @pl.loop(0, ROWS_PER_TILE // NLANE)
        def _(k):
            cv = cnt_v[pl.ds(k * NLANE, NLANE)]
            cnt_v[pl.ds(k * NLANE, NLANE)] = 1.0 / jnp.maximum(cv, 1.0)

        @pl.loop(0, NRCHUNK)
        def _(i):
            pltpu.sync_copy(table.at[pl.ds(r0 + i * RCHUNK, RCHUNK)], rowchunk)

            @pl.loop(0, RCHUNK // NLANE)
            def _(g):
                invv = cnt_v[pl.ds(i * RCHUNK + g * NLANE, NLANE)]
                for r in range(NLANE):
                    inv = jnp.full((NLANE,), invv[r], F32)
                    row = g * NLANE + r
                    for cc in range(OUT_C // NLANE):
                        rowchunk[row, pl.ds(cc * NLANE, NLANE)] = (
                            rowchunk[row, pl.ds(cc * NLANE, NLANE)] * inv)

            pltpu.sync_copy(
                rowchunk, edge_h.at[pl.ds(base + r0 + i * RCHUNK, RCHUNK)])

        pltpu.sync_copy(zeros2d.at[pl.ds(r0, ROWS_PER_TILE)],
                        table.at[pl.ds(r0, ROWS_PER_TILE)])
        pltpu.sync_copy(zeros1d.at[pl.ds(r0, ROWS_PER_TILE)],
                        bcnt.at[pl.ds(r0, ROWS_PER_TILE)])
        plsc.subcore_barrier()

        pltpu.sync_copy(ones_c.at[pl.ds(0, CC0)],
                        dcnt.at[nidx_v.at[pl.ds(0, CC0)]], add=True)
        pltpu.sync_copy(ones_c.at[pl.ds(0, CC1)],
                        dcnt.at[nidx_v.at[pl.ds(CC0, CC1)]], add=True)

        @pl.loop(0, NCHUNK)
        def _(j):
            pltpu.async_copy(
                edge_h.at[eil_v.at[pl.ds(j * CHUNK, CHUNK)]], rowbuf, sem0
            ).wait()
            pltpu.sync_copy(rowbuf,
                            table.at[nidx_v.at[pl.ds(j * CHUNK, CHUNK)]],
                            add=True)

        pltpu.async_copy(
            edge_h.at[eil_v.at[pl.ds(NCHUNK * CHUNK, TAIL)]],
            rowbuf.at[pl.ds(0, TAIL)], sem0).wait()
        pltpu.sync_copy(rowbuf.at[pl.ds(0, TAIL)],
                        table.at[nidx_v.at[pl.ds(NCHUNK * CHUNK, TAIL)]],
                        add=True)

        plsc.subcore_barrier()

        pltpu.sync_copy(table.at[pl.ds(r0, ROWS_PER_TILE)],
                        out_h.at[pl.ds(base + r0, ROWS_PER_TILE)])
        pltpu.sync_copy(dcnt.at[pl.ds(r0, ROWS_PER_TILE)],
                        dcnt_h.at[pl.ds(base + r0, ROWS_PER_TILE)])
        pltpu.sync_copy(zeros2d.at[pl.ds(r0, ROWS_PER_TILE)],
                        table.at[pl.ds(r0, ROWS_PER_TILE)])
        pltpu.sync_copy(zeros1d.at[pl.ds(r0, ROWS_PER_TILE)],
                        dcnt.at[pl.ds(r0, ROWS_PER_TILE)])
        plsc.subcore_barrier()


def _sc_hyperconv(xlf, nidx, eidx, zeros2d, zeros1d):
    mesh = plsc.VectorSubcoreMesh(core_axis_name="c", subcore_axis_name="s",
                                  num_cores=NC, num_subcores=NS)
    f = pl.kernel(
        _sc_body,
        out_type=(
            jax.ShapeDtypeStruct((NPAIRS * NPAD, OUT_C), F32),
            jax.ShapeDtypeStruct((NPAIRS * NPAD,), F32),
            jax.ShapeDtypeStruct((NPAIRS * NPAD, OUT_C), F32),
        ),
        mesh=mesh,
        scratch_types=[
            pltpu.VMEM_SHARED((NPAD, OUT_C), F32),
            pltpu.VMEM_SHARED((NPAD,), F32),
            pltpu.VMEM_SHARED((NPAD,), F32),
            pltpu.VMEM((NNZ_TILE,), I32),
            pltpu.VMEM((NNZ_TILE,), I32),
            pltpu.VMEM((CHUNK, OUT_C), F32),
            pltpu.VMEM((CC1,), F32),
            pltpu.VMEM((RCHUNK, OUT_C), F32),
            pltpu.VMEM((ROWS_PER_TILE,), F32),
            pltpu.SemaphoreType.DMA,
        ],
    )
    return f(xlf, nidx, eidx, zeros2d, zeros1d)


@jax.jit
def kernel(x, H, sequential_scene_attention, W, lin_w, bias):
    b, f, t, n = x.shape
    xt = jnp.transpose(x, (0, 2, 1, 3))
    xl = _tc_matmul(xt, sequential_scene_attention, lin_w)

    poff = (jnp.arange(NPAIRS, dtype=I32) * NPAD).reshape(NPAIRS, 1, 1)
    nidx = H[:, :, 0, :].reshape(NPAIRS, NS, NNZ_TILE) + poff
    eidx = H[:, :, 1, :].reshape(NPAIRS, NS, NNZ_TILE)
    zeros2d = jnp.zeros((NPAD, OUT_C), F32)
    zeros1d = jnp.zeros((NPAD,), F32)

    raw, dcnt, _ = _sc_hyperconv(xl.reshape(NPAIRS * NPAD, OUT_C),
                                 nidx, eidx, zeros2d, zeros1d)
    out = _tc_scale(raw.reshape(NPAIRS, NPAD, OUT_C),
                    dcnt.reshape(NPAIRS, NPAD), bias, n)
    return out.reshape(b, OUT_C, t, n)
